# speedup vs baseline: 21.9924x; 21.9924x over previous
import functools

import jax
import jax.numpy as jnp
from jax import lax
from jax.experimental import pallas as pl
from jax.experimental.pallas import tpu as pltpu
from jax.experimental.pallas import tpu_sc as plsc

N = 10000
N_PAD = 10240
D = 128
E = 320000
NC, NS, L = 2, 16, 16
NW = NC * NS
EPW = E // NW
CHUNK = 128
NFULL = EPW // CHUNK
TAIL = EPW - NFULL * CHUNK
RPT = N_PAD // NS

_mesh = plsc.VectorSubcoreMesh(core_axis_name="c", subcore_axis_name="s")


@functools.partial(
    pl.kernel,
    out_type=jax.ShapeDtypeStruct((NC, N_PAD), jnp.float32),
    mesh=_mesh,
    scratch_types=[
        pltpu.VMEM((CHUNK,), jnp.int32),
        pltpu.VMEM((TAIL,), jnp.int32),
        pltpu.VMEM((CHUNK,), jnp.float32),
        pltpu.VMEM((TAIL,), jnp.float32),
        pltpu.VMEM((RPT,), jnp.float32),
        pltpu.VMEM_SHARED((N_PAD,), jnp.float32),
    ],
)
def _deg_call(dst_hbm, out_hbm, didx, didx_t, ones_v, ones_t, zbuf, deg_sp):
    c = lax.axis_index("c")
    s = lax.axis_index("s")
    wid = s * NC + c
    ebase = wid * EPW

    zero16 = jnp.zeros((L,), jnp.float32)
    one16 = jnp.ones((L,), jnp.float32)
    for j in range(RPT // L):
        zbuf[pl.ds(j * L, L)] = zero16
    for j in range(CHUNK // L):
        ones_v[pl.ds(j * L, L)] = one16
    ones_t[...] = one16[:TAIL]

    pltpu.sync_copy(zbuf, deg_sp.at[pl.ds(s * RPT, RPT)])
    plsc.subcore_barrier()

    def body(i, _):
        off = pl.multiple_of(ebase + i * CHUNK, 16)
        pltpu.sync_copy(dst_hbm.at[pl.ds(off, CHUNK)], didx)
        pltpu.sync_copy(ones_v, deg_sp.at[didx], add=True)
        return ()

    lax.fori_loop(0, NFULL, body, ())

    off = pl.multiple_of(ebase + NFULL * CHUNK, 16)
    pltpu.sync_copy(dst_hbm.at[pl.ds(off, TAIL)], didx_t)
    pltpu.sync_copy(ones_t, deg_sp.at[didx_t], add=True)

    plsc.subcore_barrier()
    pltpu.sync_copy(deg_sp.at[pl.ds(s * RPT, RPT)],
                    out_hbm.at[c, pl.ds(s * RPT, RPT)])


@functools.partial(
    pl.kernel,
    out_type=jax.ShapeDtypeStruct((NC, N_PAD, D), jnp.float32),
    mesh=_mesh,
    scratch_types=[
        pltpu.VMEM((CHUNK,), jnp.int32),
        pltpu.VMEM((CHUNK,), jnp.int32),
        pltpu.VMEM((TAIL,), jnp.int32),
        pltpu.VMEM((TAIL,), jnp.int32),
        pltpu.VMEM((CHUNK, D), jnp.float32),
        pltpu.VMEM((TAIL, D), jnp.float32),
        pltpu.SemaphoreType.DMA,
        pltpu.VMEM_SHARED((N_PAD, D), jnp.float32),
    ],
)
def _msg_call(g_hbm, src_hbm, dst_hbm, out_hbm,
              sidx, didx, sidx_t, didx_t, rows, rows_t, sem, acc_sp):
    c = lax.axis_index("c")
    s = lax.axis_index("s")
    wid = s * NC + c
    ebase = wid * EPW

    zero16 = jnp.zeros((L,), jnp.float32)

    def zbody(t, _):
        r = t // (D // L)
        k = t % (D // L)
        rows[r, pl.ds(k * L, L)] = zero16
        return ()

    lax.fori_loop(0, CHUNK * (D // L), zbody, ())
    for j in range(RPT // CHUNK):
        pltpu.sync_copy(rows, acc_sp.at[pl.ds(s * RPT + j * CHUNK, CHUNK)])
    plsc.subcore_barrier()

    def body(i, _):
        off = pl.multiple_of(ebase + i * CHUNK, 16)
        pltpu.sync_copy(src_hbm.at[pl.ds(off, CHUNK)], sidx)
        pltpu.sync_copy(dst_hbm.at[pl.ds(off, CHUNK)], didx)
        pltpu.async_copy(g_hbm.at[sidx], rows, sem).wait()
        pltpu.sync_copy(rows, acc_sp.at[didx], add=True)
        return ()

    lax.fori_loop(0, NFULL, body, ())

    off = pl.multiple_of(ebase + NFULL * CHUNK, 16)
    pltpu.sync_copy(src_hbm.at[pl.ds(off, TAIL)], sidx_t)
    pltpu.sync_copy(dst_hbm.at[pl.ds(off, TAIL)], didx_t)
    pltpu.async_copy(g_hbm.at[sidx_t], rows_t, sem).wait()
    pltpu.sync_copy(rows_t, acc_sp.at[didx_t], add=True)

    plsc.subcore_barrier()
    pltpu.sync_copy(acc_sp.at[pl.ds(s * RPT, RPT)],
                    out_hbm.at[c, pl.ds(s * RPT, RPT)])


BLK = 1024


def _mm_body(x_ref, w_ref, ds_ref, g_ref):
    dinv = lax.rsqrt(ds_ref[...] + 1.0)
    h = jnp.dot(x_ref[...], w_ref[...], preferred_element_type=jnp.float32)
    g_ref[...] = h * dinv


def _mm_call(x, W, dsum):
    return pl.pallas_call(
        _mm_body,
        grid=(N_PAD // BLK,),
        in_specs=[
            pl.BlockSpec((BLK, D), lambda i: (i, 0)),
            pl.BlockSpec((D, D), lambda i: (0, 0)),
            pl.BlockSpec((BLK, 1), lambda i: (i, 0)),
        ],
        out_specs=pl.BlockSpec((BLK, D), lambda i: (i, 0)),
        out_shape=jax.ShapeDtypeStruct((N_PAD, D), jnp.float32),
    )(x, W, dsum)


def _out_body(acc_ref, g_ref, ds_ref, b_ref, a_ref, o_ref):
    ssum = acc_ref[0] + acc_ref[1] + g_ref[...]
    dinv = lax.rsqrt(ds_ref[...] + 1.0)
    y = ssum * dinv + b_ref[...]
    o_ref[...] = jnp.where(y >= 0, y, a_ref[0, 0] * y)


def _out_call(accp, g, dsum, b2, a2):
    return pl.pallas_call(
        _out_body,
        grid=(N_PAD // BLK,),
        in_specs=[
            pl.BlockSpec((NC, BLK, D), lambda i: (0, i, 0)),
            pl.BlockSpec((BLK, D), lambda i: (i, 0)),
            pl.BlockSpec((BLK, 1), lambda i: (i, 0)),
            pl.BlockSpec((1, D), lambda i: (0, 0)),
            pl.BlockSpec((1, 1), lambda i: (0, 0)),
        ],
        out_specs=pl.BlockSpec((BLK, D), lambda i: (i, 0)),
        out_shape=jax.ShapeDtypeStruct((N_PAD, D), jnp.float32),
    )(accp, g, dsum, b2, a2)


def kernel(x, edge_index, W, b, a):
    src = edge_index[0].astype(jnp.int32)
    dst = edge_index[1].astype(jnp.int32)
    x_pad = jnp.zeros((N_PAD, D), x.dtype).at[:N].set(x)

    degp = _deg_call(dst)
    dsum = (degp[0] + degp[1])[:, None]
    g = _mm_call(x_pad, W, dsum)
    accp = _msg_call(g, src, dst)
    out = _out_call(accp, g, dsum,
                    b.reshape(1, D).astype(jnp.float32),
                    a.reshape(1, 1).astype(jnp.float32))
    return out[:N]
